# baseline (speedup 1.0000x reference)
import dataclasses

import jax
import jax.numpy as jnp
from jax import lax
from jax.experimental import pallas as pl
from jax.experimental.pallas import tpu as pltpu
from jax.experimental.pallas import tpu_sc as plsc

_N = 10000
_E = 320000
_D_IN = 128
_D_OUT = 128

_NUM_CORES = 2
_NUM_SUBCORES = 16
_NW = _NUM_CORES * _NUM_SUBCORES
_LANES = 16
_EDGE_BLOCK = 1280


def _sc_segment_sum(ei2d, ev2d):
    mesh = plsc.VectorSubcoreMesh(core_axis_name="c", subcore_axis_name="s")
    cp = pltpu.CompilerParams()
    if "needs_layout_passes" in pltpu.CompilerParams.__dataclass_fields__:
        cp = dataclasses.replace(cp, needs_layout_passes=False)

    @pl.kernel(
        out_type=jax.ShapeDtypeStruct((_NW, _N), jnp.float32),
        mesh=mesh,
        scratch_types=[pltpu.VMEM((_N,), jnp.float32)],
        compiler_params=cp,
    )
    def seg_sum_kernel(ei_hbm, ev_hbm, out_hbm, acc):
        @pl.loop(0, _N, step=_LANES, unroll=8)
        def _(i):
            acc.at[pl.ds(i, _LANES)][...] = jnp.zeros((_LANES,), jnp.float32)

        def body(i_vmem, v_vmem):
            @pl.loop(0, _EDGE_BLOCK, step=_LANES, unroll=8)
            def _(c):
                idx = i_vmem.at[0, pl.ds(c, _LANES)][...]
                val = v_vmem.at[0, pl.ds(c, _LANES)][...]
                plsc.addupdate_scatter(acc, [idx], val)

        pltpu.emit_pipeline(
            body,
            grid=(_E // _EDGE_BLOCK,),
            in_specs=[
                pl.BlockSpec((1, _EDGE_BLOCK), lambda i: (1, i)),
                pl.BlockSpec((1, _EDGE_BLOCK), lambda i: (0, i)),
            ],
            out_specs=[],
            core_axis_name=("c", "s"),
            dimension_semantics=(pltpu.PARALLEL,),
        )(ei_hbm, ev_hbm)

        wid = lax.axis_index("s") * _NUM_CORES + lax.axis_index("c")
        pltpu.sync_copy(acc, out_hbm.at[wid])

    return seg_sum_kernel(ei2d, ev2d)


def _tc_mm_body(h_ref, w_ref, p_ref):
    p_ref[...] = jax.lax.dot_general(
        h_ref[...], w_ref[...], (((1,), (1,)), ((), ())),
        preferred_element_type=jnp.float32)


def _tc_mm(h, w):
    return pl.pallas_call(
        _tc_mm_body,
        out_shape=jax.ShapeDtypeStruct((_N, _D_OUT), jnp.float32),
    )(h, w)


def _tc_scale_body(p_ref, sp_ref, norm_ref, b_ref, o_ref):
    ones = jnp.ones((_NW, 1), jnp.float32)
    s = jax.lax.dot_general(
        sp_ref[...], ones, (((0,), (0,)), ((), ())),
        preferred_element_type=jnp.float32)
    nrm = norm_ref[...]
    o_ref[...] = p_ref[...] * (s * nrm) + b_ref[...] * nrm


def _tc_scale(p, s_part, norm, b2d):
    return pl.pallas_call(
        _tc_scale_body,
        out_shape=jax.ShapeDtypeStruct((_N, _D_OUT), jnp.float32),
        input_output_aliases={0: 0},
    )(p, s_part, norm, b2d)


def kernel(h, e_h, norm, edge_index, W, b):
    ev = e_h.reshape(1, _E)
    s_part = _sc_segment_sum(edge_index, ev)
    p = _tc_mm(h, W)
    return _tc_scale(p, s_part, norm, b.reshape(1, _D_OUT))
